# baseline (speedup 1.0000x reference)
import functools

import jax
import jax.numpy as jnp
from jax import lax
from jax.experimental import pallas as pl
from jax.experimental.pallas import tpu as pltpu
from jax.experimental.pallas import tpu_sc as plsc

U_NUM = 50000
I_NUM = 10000
U_PAD = 50176
I_PAD = 10240
HID = 128
N_EDGES = 600000
NSUB = 16
NCORE = 2
NCHUNK_U = 4

KB = 8
NBLK = 4864
E_PAD = NBLK * 128
BLK_U = NBLK // NSUB
NBATCH_U = BLK_U // KB
BLK_I = NBLK // (NSUB * NCORE)
NBATCH_I = BLK_I // KB

CCAP = KB * 128 + 128
GROW = 128


def _make_spmm_user():
    chunk_rows = U_PAD // NCHUNK_U
    rows_per_sub = chunk_rows // NSUB
    zrows = 16
    nz = rows_per_sub // zrows
    mesh = plsc.VectorSubcoreMesh(core_axis_name="c", subcore_axis_name="s")

    @functools.partial(
        pl.kernel,
        mesh=mesh,
        compiler_params=pltpu.CompilerParams(needs_layout_passes=False),
        out_type=jax.ShapeDtypeStruct((U_PAD, HID), jnp.float32),
        scratch_types=[
            pltpu.VMEM_SHARED((chunk_rows, HID), jnp.float32),
            pltpu.VMEM((KB, 128), jnp.int32),
            pltpu.VMEM((KB, 128), jnp.int32),
            pltpu.VMEM((KB, 128), jnp.float32),
            pltpu.VMEM((CCAP,), jnp.int32),
            pltpu.VMEM((CCAP,), jnp.int32),
            pltpu.VMEM((CCAP,), jnp.float32),
            pltpu.VMEM((1, GROW), jnp.int32),
            pltpu.VMEM((1, GROW), jnp.int32),
            pltpu.VMEM((GROW, HID), jnp.float32),
            pltpu.VMEM((zrows, HID), jnp.float32),
            pltpu.SemaphoreType.DMA,
        ],
    )
    def spmm(src_hbm, esrc_hbm, edst_hbm, evals_hbm, out_hbm,
             acc, bsrc, bdst, bval, csrc, cdst, cval,
             sstage, dstage, grow, zbuf, sem):
        core = lax.axis_index("c")
        sub = lax.axis_index("s")
        blk0 = sub * BLK_U
        row0 = sub * rows_per_sub
        lane = lax.iota(jnp.int32, 16)

        for i in range(zrows):
            for h in range(HID // 16):
                zbuf[i, pl.ds(h * 16, 16)] = jnp.zeros((16,), jnp.float32)

        for chunk in range(NCHUNK_U // NCORE):
            cblk = core * (NCHUNK_U // NCORE) + chunk
            lo = cblk * chunk_rows
            hi = lo + chunk_rows

            def zc_body(i, c):
                pltpu.sync_copy(zbuf, acc.at[pl.ds(row0 + i * zrows, zrows)])
                return c
            lax.fori_loop(0, nz, zc_body, 0)
            plsc.subcore_barrier()

            def batch_body(b, c):
                base = blk0 + b * KB
                pltpu.sync_copy(esrc_hbm.at[pl.ds(base, KB)], bsrc)
                pltpu.sync_copy(edst_hbm.at[pl.ds(base, KB)], bdst)
                pltpu.sync_copy(evals_hbm.at[pl.ds(base, KB)], bval)

                @plsc.parallel_loop(0, KB * 8, unroll=2,
                                    carry=jnp.int32(0))
                def cnt(g, cnt):
                    r = g // 8
                    o = (g % 8) * 16
                    dv = bdst[r, pl.ds(o, 16)]
                    sv = bsrc[r, pl.ds(o, 16)]
                    vv = bval[r, pl.ds(o, 16)]
                    m = (dv >= lo) & (dv < hi)
                    cs = plsc.cumsum(jnp.where(m, 1, 0))
                    pos = cs + (cnt - 1)
                    plsc.store_scatter(cdst, [pos], dv - lo, mask=m)
                    plsc.store_scatter(csrc, [pos], sv, mask=m)
                    plsc.store_scatter(cval, [pos], vv, mask=m)
                    return cnt + cs[15]

                for t in range(128 // 16):
                    cdst[pl.ds(cnt + t * 16, 16)] = lane + (t * 16)
                    csrc[pl.ds(cnt + t * 16, 16)] = lane + (t * 16)
                    cval[pl.ds(cnt + t * 16, 16)] = jnp.zeros((16,),
                                                              jnp.float32)

                nb = (cnt + GROW - 1) // GROW

                def proc_body(q, c2):
                    q0 = q * GROW
                    for t in range(GROW // 16):
                        sstage[0, pl.ds(t * 16, 16)] = (
                            csrc[pl.ds(q0 + t * 16, 16)])
                        dstage[0, pl.ds(t * 16, 16)] = (
                            cdst[pl.ds(q0 + t * 16, 16)])
                    pltpu.async_copy(
                        src_hbm.at[sstage.at[0]], grow, sem).wait()

                    @plsc.parallel_loop(0, GROW // 16, unroll=4)
                    def _(g2):
                        vv = cval[pl.ds(q0 + g2 * 16, 16)]
                        for i in range(16):
                            v = vv[i]
                            e = g2 * 16 + i
                            for h in range(HID // 16):
                                grow[e, pl.ds(h * 16, 16)] = (
                                    grow[e, pl.ds(h * 16, 16)] * v)

                    pltpu.sync_copy(grow, acc.at[dstage.at[0]], add=True)
                    return c2
                lax.fori_loop(0, nb, proc_body, 0)
                return c
            lax.fori_loop(0, NBATCH_U, batch_body, 0)
            plsc.subcore_barrier()

            pltpu.sync_copy(
                acc.at[pl.ds(row0, rows_per_sub)],
                out_hbm.at[pl.ds(lo + row0, rows_per_sub)])
            plsc.subcore_barrier()

    return spmm


def _make_spmm_item():
    rows_per_sub = I_PAD // NSUB
    zrows = 64
    nz = rows_per_sub // zrows
    mesh = plsc.VectorSubcoreMesh(core_axis_name="c", subcore_axis_name="s")

    @functools.partial(
        pl.kernel,
        mesh=mesh,
        compiler_params=pltpu.CompilerParams(needs_layout_passes=False),
        out_type=jax.ShapeDtypeStruct((NCORE * I_PAD, HID), jnp.float32),
        scratch_types=[
            pltpu.VMEM_SHARED((I_PAD, HID), jnp.float32),
            pltpu.VMEM((KB, 128), jnp.int32),
            pltpu.VMEM((KB, 128), jnp.int32),
            pltpu.VMEM((KB, 128), jnp.float32),
            pltpu.VMEM((GROW, HID), jnp.float32),
            pltpu.VMEM((GROW, HID), jnp.float32),
            pltpu.VMEM((zrows, HID), jnp.float32),
            pltpu.SemaphoreType.DMA,
        ],
    )
    def spmm(src_hbm, esrc_hbm, edst_hbm, evals_hbm, out_hbm,
             acc, bsrc, bdst, bval, grow0, grow1, zbuf, sem):
        core = lax.axis_index("c")
        sub = lax.axis_index("s")
        wid = core * NSUB + sub
        blk0 = wid * BLK_I
        row0 = sub * rows_per_sub

        for i in range(zrows):
            for h in range(HID // 16):
                zbuf[i, pl.ds(h * 16, 16)] = jnp.zeros((16,), jnp.float32)

        def zc_body(i, c):
            pltpu.sync_copy(zbuf, acc.at[pl.ds(row0 + i * zrows, zrows)])
            return c
        lax.fori_loop(0, nz, zc_body, 0)
        plsc.subcore_barrier()

        grows = (grow0, grow1)

        def batch_body(b, c):
            base = blk0 + b * KB
            pltpu.sync_copy(esrc_hbm.at[pl.ds(base, KB)], bsrc)
            pltpu.sync_copy(edst_hbm.at[pl.ds(base, KB)], bdst)
            pltpu.sync_copy(evals_hbm.at[pl.ds(base, KB)], bval)

            pltpu.async_copy(src_hbm.at[bsrc.at[0]], grows[0], sem)
            for k in range(KB):
                g = grows[k % 2]
                pltpu.make_async_copy(
                    src_hbm.at[pl.ds(0, GROW)], g, sem).wait()
                if k + 1 < KB:
                    pltpu.async_copy(
                        src_hbm.at[bsrc.at[k + 1]], grows[(k + 1) % 2], sem)

                @plsc.parallel_loop(0, GROW // 16, unroll=4)
                def _(g2, _g=g, _k=k):
                    vv = bval[_k, pl.ds(g2 * 16, 16)]
                    for i in range(16):
                        v = vv[i]
                        e = g2 * 16 + i
                        for h in range(HID // 16):
                            _g[e, pl.ds(h * 16, 16)] = (
                                _g[e, pl.ds(h * 16, 16)] * v)

                pltpu.sync_copy(g, acc.at[bdst.at[k]], add=True)
            return c
        lax.fori_loop(0, NBATCH_I, batch_body, 0)
        plsc.subcore_barrier()

        pltpu.sync_copy(
            acc.at[pl.ds(row0, rows_per_sub)],
            out_hbm.at[pl.ds(core * I_PAD + row0, rows_per_sub)])
        plsc.subcore_barrier()

    return spmm


_spmm_to_user = _make_spmm_user()
_spmm_to_item = _make_spmm_item()


def _dense_user(x, w, prev, r):
    def body(*refs):
        if prev is None:
            x_ref, w_ref, o_ref = refs
        else:
            x_ref, w_ref, p_ref, o_ref = refs
        y = jax.nn.sigmoid(
            jnp.dot(x_ref[:, :], w_ref[:, :],
                    preferred_element_type=jnp.float32))
        if prev is not None:
            y = (y + p_ref[:, :]) * 0.5
        o_ref[:, :] = y

    in_specs = [
        pl.BlockSpec((r, HID), lambda i: (i, 0)),
        pl.BlockSpec((HID, HID), lambda i: (0, 0)),
    ]
    args = [x, w]
    if prev is not None:
        in_specs.append(pl.BlockSpec((r, HID), lambda i: (i, 0)))
        args.append(prev)
    return pl.pallas_call(
        body,
        grid=(U_NUM // r,),
        in_specs=in_specs,
        out_specs=pl.BlockSpec((r, HID), lambda i: (i, 0)),
        out_shape=jax.ShapeDtypeStruct((U_NUM, HID), jnp.float32),
    )(*args)


def _dense_item(x2, w, prev, r):
    def body(*refs):
        if prev is None:
            a_ref, b_ref, w_ref, o_ref = refs
        else:
            a_ref, b_ref, w_ref, p_ref, o_ref = refs
        x = a_ref[0] + b_ref[0]
        y = jax.nn.sigmoid(
            jnp.dot(x, w_ref[:, :], preferred_element_type=jnp.float32))
        if prev is not None:
            y = (y + p_ref[:, :]) * 0.5
        o_ref[:, :] = y

    in_specs = [
        pl.BlockSpec((1, r, HID), lambda i: (0, i, 0)),
        pl.BlockSpec((1, r, HID), lambda i: (1, i, 0)),
        pl.BlockSpec((HID, HID), lambda i: (0, 0)),
    ]
    args = [x2, x2, w]
    if prev is not None:
        in_specs.append(pl.BlockSpec((r, HID), lambda i: (i, 0)))
        args.append(prev)
    return pl.pallas_call(
        body,
        grid=(I_NUM // r,),
        in_specs=in_specs,
        out_specs=pl.BlockSpec((r, HID), lambda i: (i, 0)),
        out_shape=jax.ShapeDtypeStruct((I_NUM, HID), jnp.float32),
    )(*args)


def kernel(user_embedding, item_embedding, edge_user, edge_item, edge_vals,
           u_w0, i_w0, u_w1, i_w1):
    pad = E_PAD - N_EDGES
    ar = jnp.arange(pad, dtype=jnp.int32)
    eu = jnp.concatenate([edge_user.astype(jnp.int32), ar % U_NUM])
    ei = jnp.concatenate([edge_item.astype(jnp.int32), ar % I_NUM])
    ev = jnp.concatenate([edge_vals, jnp.zeros((pad,), jnp.float32)])
    eu_b = eu.reshape(NBLK, 128)
    ei_b = ei.reshape(NBLK, 128)
    ev_b = ev.reshape(NBLK, 128)

    r_u, r_i = 2000, 2000

    up0 = _spmm_to_user(item_embedding, ei_b, eu_b, ev_b)
    ip0 = _spmm_to_item(up0, eu_b, ei_b, ev_b).reshape(NCORE, I_PAD, HID)
    u_emb0 = _dense_user(up0, u_w0, None, r_u)
    i_emb0 = _dense_item(ip0, i_w0, None, r_i)
    up1 = _spmm_to_user(i_emb0, ei_b, eu_b, ev_b)
    ip1 = _spmm_to_item(up1, eu_b, ei_b, ev_b).reshape(NCORE, I_PAD, HID)
    user_out = _dense_user(up1, u_w1, u_emb0, r_u)
    item_out = _dense_item(ip1, i_w1, i_emb0, r_i)
    return (user_out, item_out)


# speedup vs baseline: 1.0271x; 1.0271x over previous
import functools

import jax
import jax.numpy as jnp
from jax import lax
from jax.experimental import pallas as pl
from jax.experimental.pallas import tpu as pltpu
from jax.experimental.pallas import tpu_sc as plsc

U_NUM = 50000
I_NUM = 10000
U_PAD = 50176
I_PAD = 10240
HID = 128
N_EDGES = 600000
NSUB = 16
NCORE = 2
NCHUNK_U = 4

KB = 8
NBLK = 4864
E_PAD = NBLK * 128
BLK_U = NBLK // NSUB
NBATCH_U = BLK_U // KB
BLK_I = NBLK // (NSUB * NCORE)
NBATCH_I = BLK_I // KB

CCAP = KB * 128 + 128
GROW = 128


def _make_spmm_user():
    chunk_rows = U_PAD // NCHUNK_U
    rows_per_sub = chunk_rows // NSUB
    zrows = 16
    nz = rows_per_sub // zrows
    mesh = plsc.VectorSubcoreMesh(core_axis_name="c", subcore_axis_name="s")

    @functools.partial(
        pl.kernel,
        mesh=mesh,
        compiler_params=pltpu.CompilerParams(needs_layout_passes=False),
        out_type=jax.ShapeDtypeStruct((U_PAD, HID), jnp.float32),
        scratch_types=[
            pltpu.VMEM_SHARED((chunk_rows, HID), jnp.float32),
            pltpu.VMEM((KB, 128), jnp.int32),
            pltpu.VMEM((KB, 128), jnp.int32),
            pltpu.VMEM((KB, 128), jnp.float32),
            pltpu.VMEM((CCAP,), jnp.int32),
            pltpu.VMEM((CCAP,), jnp.int32),
            pltpu.VMEM((CCAP,), jnp.float32),
            pltpu.VMEM((1, GROW), jnp.int32),
            pltpu.VMEM((1, GROW), jnp.int32),
            pltpu.VMEM((GROW, HID), jnp.float32),
            pltpu.VMEM((zrows, HID), jnp.float32),
            pltpu.SemaphoreType.DMA,
        ],
    )
    def spmm(src_hbm, esrc_hbm, edst_hbm, evals_hbm, out_hbm,
             acc, bsrc, bdst, bval, csrc, cdst, cval,
             sstage, dstage, grow, zbuf, sem):
        core = lax.axis_index("c")
        sub = lax.axis_index("s")
        blk0 = sub * BLK_U
        row0 = sub * rows_per_sub
        lane = lax.iota(jnp.int32, 16)

        for i in range(zrows):
            for h in range(HID // 16):
                zbuf[i, pl.ds(h * 16, 16)] = jnp.zeros((16,), jnp.float32)

        for chunk in range(NCHUNK_U // NCORE):
            cblk = core * (NCHUNK_U // NCORE) + chunk
            lo = cblk * chunk_rows
            hi = lo + chunk_rows

            def zc_body(i, c):
                pltpu.sync_copy(zbuf, acc.at[pl.ds(row0 + i * zrows, zrows)])
                return c
            lax.fori_loop(0, nz, zc_body, 0)
            plsc.subcore_barrier()

            def batch_body(b, c):
                base = blk0 + b * KB
                pltpu.sync_copy(esrc_hbm.at[pl.ds(base, KB)], bsrc)
                pltpu.sync_copy(edst_hbm.at[pl.ds(base, KB)], bdst)
                pltpu.sync_copy(evals_hbm.at[pl.ds(base, KB)], bval)

                @plsc.parallel_loop(0, KB * 8, unroll=2,
                                    carry=jnp.int32(0))
                def cnt(g, cnt):
                    r = g // 8
                    o = (g % 8) * 16
                    dv = bdst[r, pl.ds(o, 16)]
                    sv = bsrc[r, pl.ds(o, 16)]
                    vv = bval[r, pl.ds(o, 16)]
                    m = (dv >= lo) & (dv < hi)
                    cs = plsc.cumsum(jnp.where(m, 1, 0))
                    pos = cs + (cnt - 1)
                    plsc.store_scatter(cdst, [pos], dv - lo, mask=m)
                    plsc.store_scatter(csrc, [pos], sv, mask=m)
                    plsc.store_scatter(cval, [pos], vv, mask=m)
                    return cnt + cs[15]

                for t in range(128 // 16):
                    cdst[pl.ds(cnt + t * 16, 16)] = lane + (t * 16)
                    csrc[pl.ds(cnt + t * 16, 16)] = lane + (t * 16)
                    cval[pl.ds(cnt + t * 16, 16)] = jnp.zeros((16,),
                                                              jnp.float32)

                nb = (cnt + GROW - 1) // GROW

                def proc_body(q, c2):
                    q0 = q * GROW
                    for t in range(GROW // 16):
                        sstage[0, pl.ds(t * 16, 16)] = (
                            csrc[pl.ds(q0 + t * 16, 16)])
                        dstage[0, pl.ds(t * 16, 16)] = (
                            cdst[pl.ds(q0 + t * 16, 16)])
                    pltpu.async_copy(
                        src_hbm.at[sstage.at[0]], grow, sem).wait()

                    @plsc.parallel_loop(0, GROW // 16, unroll=2)
                    def _(g2):
                        vv = cval[pl.ds(q0 + g2 * 16, 16)]
                        for i in range(16):
                            v = vv[i]
                            e = g2 * 16 + i
                            for h in range(HID // 16):
                                grow[e, pl.ds(h * 16, 16)] = (
                                    grow[e, pl.ds(h * 16, 16)] * v)

                    pltpu.sync_copy(grow, acc.at[dstage.at[0]], add=True)
                    return c2
                lax.fori_loop(0, nb, proc_body, 0)
                return c
            lax.fori_loop(0, NBATCH_U, batch_body, 0)
            plsc.subcore_barrier()

            pltpu.sync_copy(
                acc.at[pl.ds(row0, rows_per_sub)],
                out_hbm.at[pl.ds(lo + row0, rows_per_sub)])
            plsc.subcore_barrier()

    return spmm


def _make_spmm_item():
    rows_per_sub = I_PAD // NSUB
    zrows = 64
    nz = rows_per_sub // zrows
    mesh = plsc.VectorSubcoreMesh(core_axis_name="c", subcore_axis_name="s")

    @functools.partial(
        pl.kernel,
        mesh=mesh,
        compiler_params=pltpu.CompilerParams(needs_layout_passes=False),
        out_type=jax.ShapeDtypeStruct((NCORE * I_PAD, HID), jnp.float32),
        scratch_types=[
            pltpu.VMEM_SHARED((I_PAD, HID), jnp.float32),
            pltpu.VMEM((KB, 128), jnp.int32),
            pltpu.VMEM((KB, 128), jnp.int32),
            pltpu.VMEM((KB, 128), jnp.float32),
            pltpu.VMEM((GROW, HID), jnp.float32),
            pltpu.VMEM((GROW, HID), jnp.float32),
            pltpu.VMEM((zrows, HID), jnp.float32),
            pltpu.SemaphoreType.DMA,
        ],
    )
    def spmm(src_hbm, esrc_hbm, edst_hbm, evals_hbm, out_hbm,
             acc, bsrc, bdst, bval, grow0, grow1, zbuf, sem):
        core = lax.axis_index("c")
        sub = lax.axis_index("s")
        wid = core * NSUB + sub
        blk0 = wid * BLK_I
        row0 = sub * rows_per_sub

        for i in range(zrows):
            for h in range(HID // 16):
                zbuf[i, pl.ds(h * 16, 16)] = jnp.zeros((16,), jnp.float32)

        def zc_body(i, c):
            pltpu.sync_copy(zbuf, acc.at[pl.ds(row0 + i * zrows, zrows)])
            return c
        lax.fori_loop(0, nz, zc_body, 0)
        plsc.subcore_barrier()

        grows = (grow0, grow1)

        def batch_body(b, c):
            base = blk0 + b * KB
            pltpu.sync_copy(esrc_hbm.at[pl.ds(base, KB)], bsrc)
            pltpu.sync_copy(edst_hbm.at[pl.ds(base, KB)], bdst)
            pltpu.sync_copy(evals_hbm.at[pl.ds(base, KB)], bval)

            pltpu.async_copy(src_hbm.at[bsrc.at[0]], grows[0], sem)
            for k in range(KB):
                g = grows[k % 2]
                pltpu.make_async_copy(
                    src_hbm.at[pl.ds(0, GROW)], g, sem).wait()
                if k + 1 < KB:
                    pltpu.async_copy(
                        src_hbm.at[bsrc.at[k + 1]], grows[(k + 1) % 2], sem)

                @plsc.parallel_loop(0, GROW // 16, unroll=2)
                def _(g2, _g=g, _k=k):
                    vv = bval[_k, pl.ds(g2 * 16, 16)]
                    for i in range(16):
                        v = vv[i]
                        e = g2 * 16 + i
                        for h in range(HID // 16):
                            _g[e, pl.ds(h * 16, 16)] = (
                                _g[e, pl.ds(h * 16, 16)] * v)

                pltpu.sync_copy(g, acc.at[bdst.at[k]], add=True)
            return c
        lax.fori_loop(0, NBATCH_I, batch_body, 0)
        plsc.subcore_barrier()

        pltpu.sync_copy(
            acc.at[pl.ds(row0, rows_per_sub)],
            out_hbm.at[pl.ds(core * I_PAD + row0, rows_per_sub)])
        plsc.subcore_barrier()

    return spmm


_spmm_to_user = _make_spmm_user()
_spmm_to_item = _make_spmm_item()


def _dense_user(x, w, prev, r):
    def body(*refs):
        if prev is None:
            x_ref, w_ref, o_ref = refs
        else:
            x_ref, w_ref, p_ref, o_ref = refs
        y = jax.nn.sigmoid(
            jnp.dot(x_ref[:, :], w_ref[:, :],
                    preferred_element_type=jnp.float32))
        if prev is not None:
            y = (y + p_ref[:, :]) * 0.5
        o_ref[:, :] = y

    in_specs = [
        pl.BlockSpec((r, HID), lambda i: (i, 0)),
        pl.BlockSpec((HID, HID), lambda i: (0, 0)),
    ]
    args = [x, w]
    if prev is not None:
        in_specs.append(pl.BlockSpec((r, HID), lambda i: (i, 0)))
        args.append(prev)
    return pl.pallas_call(
        body,
        grid=(U_NUM // r,),
        in_specs=in_specs,
        out_specs=pl.BlockSpec((r, HID), lambda i: (i, 0)),
        out_shape=jax.ShapeDtypeStruct((U_NUM, HID), jnp.float32),
    )(*args)


def _dense_item(x2, w, prev, r):
    def body(*refs):
        if prev is None:
            a_ref, b_ref, w_ref, o_ref = refs
        else:
            a_ref, b_ref, w_ref, p_ref, o_ref = refs
        x = a_ref[0] + b_ref[0]
        y = jax.nn.sigmoid(
            jnp.dot(x, w_ref[:, :], preferred_element_type=jnp.float32))
        if prev is not None:
            y = (y + p_ref[:, :]) * 0.5
        o_ref[:, :] = y

    in_specs = [
        pl.BlockSpec((1, r, HID), lambda i: (0, i, 0)),
        pl.BlockSpec((1, r, HID), lambda i: (1, i, 0)),
        pl.BlockSpec((HID, HID), lambda i: (0, 0)),
    ]
    args = [x2, x2, w]
    if prev is not None:
        in_specs.append(pl.BlockSpec((r, HID), lambda i: (i, 0)))
        args.append(prev)
    return pl.pallas_call(
        body,
        grid=(I_NUM // r,),
        in_specs=in_specs,
        out_specs=pl.BlockSpec((r, HID), lambda i: (i, 0)),
        out_shape=jax.ShapeDtypeStruct((I_NUM, HID), jnp.float32),
    )(*args)


def kernel(user_embedding, item_embedding, edge_user, edge_item, edge_vals,
           u_w0, i_w0, u_w1, i_w1):
    pad = E_PAD - N_EDGES
    ar = jnp.arange(pad, dtype=jnp.int32)
    eu = jnp.concatenate([edge_user.astype(jnp.int32), ar % U_NUM])
    ei = jnp.concatenate([edge_item.astype(jnp.int32), ar % I_NUM])
    ev = jnp.concatenate([edge_vals, jnp.zeros((pad,), jnp.float32)])
    eu_b = eu.reshape(NBLK, 128)
    ei_b = ei.reshape(NBLK, 128)
    ev_b = ev.reshape(NBLK, 128)

    r_u, r_i = 2000, 2000

    up0 = _spmm_to_user(item_embedding, ei_b, eu_b, ev_b)
    ip0 = _spmm_to_item(up0, eu_b, ei_b, ev_b).reshape(NCORE, I_PAD, HID)
    u_emb0 = _dense_user(up0, u_w0, None, r_u)
    i_emb0 = _dense_item(ip0, i_w0, None, r_i)
    up1 = _spmm_to_user(i_emb0, ei_b, eu_b, ev_b)
    ip1 = _spmm_to_item(up1, eu_b, ei_b, ev_b).reshape(NCORE, I_PAD, HID)
    user_out = _dense_user(up1, u_w1, u_emb0, r_u)
    item_out = _dense_item(ip1, i_w1, i_emb0, r_i)
    return (user_out, item_out)
